# Initial kernel scaffold; baseline (speedup 1.0000x reference)
import jax
import jax.numpy as jnp
from jax.experimental import pallas as pl


def kernel(z_e, codebook):
    raise NotImplementedError("write your pallas kernel here")


# speedup vs baseline: 1.5854x; 1.5854x over previous
import jax
import jax.numpy as jnp
from jax.experimental import pallas as pl
from jax.experimental.pallas import tpu as pltpu

_NCODES = 1024
_D = 64
_BLK = 1024


def _vq_body(z_ref, cb_ref, zq_ref, idx_ref, loss_ref, perp_ref,
             counts_ref, losssum_ref):
    step = pl.program_id(0)
    nsteps = pl.num_programs(0)
    ntok = nsteps * _BLK

    z = z_ref[...]
    cb = cb_ref[...]

    z2 = jnp.sum(z * z, axis=1, keepdims=True)
    c2 = jnp.sum(cb * cb, axis=1)
    m = jax.lax.dot_general(z, cb, (((1,), (1,)), ((), ())),
                            preferred_element_type=jnp.float32)
    d = (z2 + c2[None, :]) - 2.0 * m

    dmin = jnp.min(d, axis=1, keepdims=True)
    iota = jax.lax.broadcasted_iota(jnp.int32, (_BLK, _NCODES), 1)
    idx = jnp.min(jnp.where(d == dmin, iota, _NCODES), axis=1,
                  keepdims=True)
    idx_ref[...] = idx

    onehot = (iota == idx).astype(jnp.float32)
    zq = jax.lax.dot_general(onehot, cb, (((1,), (0,)), ((), ())),
                             preferred_element_type=jnp.float32)
    zq_ref[...] = zq

    diff = zq - z
    part_loss = jnp.sum(diff * diff)
    part_counts = jax.lax.dot_general(
        jnp.ones((1, _BLK), jnp.float32), onehot, (((1,), (0,)), ((), ())),
        preferred_element_type=jnp.float32)

    @pl.when(step == 0)
    def _init():
        counts_ref[...] = jnp.zeros_like(counts_ref)
        losssum_ref[...] = jnp.zeros_like(losssum_ref)

    counts_ref[...] += part_counts
    losssum_ref[...] = losssum_ref[...] + part_loss

    @pl.when(step == nsteps - 1)
    def _fin():
        loss_ref[...] = losssum_ref[...] / (ntok * _D)
        p = counts_ref[...] / ntok
        s = jnp.sum(p * jnp.log(p + 1e-10), axis=1, keepdims=True)
        perp_ref[...] = jnp.exp(-s)


def kernel(z_e, codebook):
    shape = z_e.shape
    flat = z_e.reshape(-1, _D)
    ntok = flat.shape[0]
    grid = ntok // _BLK

    zq, idx, loss, perp = pl.pallas_call(
        _vq_body,
        grid=(grid,),
        in_specs=[
            pl.BlockSpec((_BLK, _D), lambda i: (i, 0)),
            pl.BlockSpec((_NCODES, _D), lambda i: (0, 0)),
        ],
        out_specs=[
            pl.BlockSpec((_BLK, _D), lambda i: (i, 0)),
            pl.BlockSpec((_BLK, 1), lambda i: (i, 0)),
            pl.BlockSpec((1, 1), lambda i: (0, 0)),
            pl.BlockSpec((1, 1), lambda i: (0, 0)),
        ],
        out_shape=[
            jax.ShapeDtypeStruct((ntok, _D), jnp.float32),
            jax.ShapeDtypeStruct((ntok, 1), jnp.int32),
            jax.ShapeDtypeStruct((1, 1), jnp.float32),
            jax.ShapeDtypeStruct((1, 1), jnp.float32),
        ],
        scratch_shapes=[
            pltpu.VMEM((1, _NCODES), jnp.float32),
            pltpu.VMEM((1, 1), jnp.float32),
        ],
    )(flat, codebook)

    z_q_st = zq.reshape(shape)
    indices_r = idx[:, 0].reshape(shape[:-1])
    loss_s = loss[0, 0]
    return (z_q_st, indices_r, loss_s, loss_s, perp[0, 0])


# speedup vs baseline: 1.7131x; 1.0806x over previous
import jax
import jax.numpy as jnp
from jax.experimental import pallas as pl
from jax.experimental.pallas import tpu as pltpu

_NCODES = 1024
_D = 64
_BLK = 2048


def _vq_body(z_ref, cb_ref, zq_ref, idx_ref, loss_ref, perp_ref,
             counts_ref, losssum_ref):
    step = pl.program_id(0)
    nsteps = pl.num_programs(0)
    ntok = nsteps * _BLK

    z = z_ref[...]
    cb = cb_ref[...]

    z2 = jnp.sum(z * z, axis=1, keepdims=True)
    c2 = jnp.sum(cb * cb, axis=1)
    m = jax.lax.dot_general(z, cb, (((1,), (1,)), ((), ())),
                            preferred_element_type=jnp.float32)
    d = (z2 + c2[None, :]) - 2.0 * m

    dmin = jnp.min(d, axis=1, keepdims=True)
    iota = jax.lax.broadcasted_iota(jnp.int32, (_BLK, _NCODES), 1)
    idx = jnp.min(jnp.where(d == dmin, iota, _NCODES), axis=1,
                  keepdims=True)
    idx_ref[...] = idx

    onehot = (iota == idx).astype(jnp.float32)
    zq = jax.lax.dot_general(onehot, cb, (((1,), (0,)), ((), ())),
                             preferred_element_type=jnp.float32)
    zq_ref[...] = zq

    diff = zq - z
    part_loss = jnp.sum(diff * diff)
    part_counts = jax.lax.dot_general(
        jnp.ones((1, _BLK), jnp.float32), onehot, (((1,), (0,)), ((), ())),
        preferred_element_type=jnp.float32)

    @pl.when(step == 0)
    def _init():
        counts_ref[...] = jnp.zeros_like(counts_ref)
        losssum_ref[...] = jnp.zeros_like(losssum_ref)

    counts_ref[...] += part_counts
    losssum_ref[...] = losssum_ref[...] + part_loss

    @pl.when(step == nsteps - 1)
    def _fin():
        loss_ref[...] = losssum_ref[...] / (ntok * _D)
        p = counts_ref[...] / ntok
        s = jnp.sum(p * jnp.log(p + 1e-10), axis=1, keepdims=True)
        perp_ref[...] = jnp.exp(-s)


def kernel(z_e, codebook):
    shape = z_e.shape
    flat = z_e.reshape(-1, _D)
    ntok = flat.shape[0]
    grid = ntok // _BLK

    zq, idx, loss, perp = pl.pallas_call(
        _vq_body,
        grid=(grid,),
        in_specs=[
            pl.BlockSpec((_BLK, _D), lambda i: (i, 0)),
            pl.BlockSpec((_NCODES, _D), lambda i: (0, 0)),
        ],
        out_specs=[
            pl.BlockSpec((_BLK, _D), lambda i: (i, 0)),
            pl.BlockSpec((_BLK, 1), lambda i: (i, 0)),
            pl.BlockSpec((1, 1), lambda i: (0, 0)),
            pl.BlockSpec((1, 1), lambda i: (0, 0)),
        ],
        out_shape=[
            jax.ShapeDtypeStruct((ntok, _D), jnp.float32),
            jax.ShapeDtypeStruct((ntok, 1), jnp.int32),
            jax.ShapeDtypeStruct((1, 1), jnp.float32),
            jax.ShapeDtypeStruct((1, 1), jnp.float32),
        ],
        scratch_shapes=[
            pltpu.VMEM((1, _NCODES), jnp.float32),
            pltpu.VMEM((1, 1), jnp.float32),
        ],
    )(flat, codebook)

    z_q_st = zq.reshape(shape)
    indices_r = idx[:, 0].reshape(shape[:-1])
    loss_s = loss[0, 0]
    return (z_q_st, indices_r, loss_s, loss_s, perp[0, 0])


# speedup vs baseline: 1.7719x; 1.0343x over previous
import jax
import jax.numpy as jnp
from jax.experimental import pallas as pl
from jax.experimental.pallas import tpu as pltpu

_NCODES = 1024
_D = 64
_BLK = 4096


def _vq_body(z_ref, cb_ref, zq_ref, idx_ref, loss_ref, perp_ref,
             counts_ref, losssum_ref):
    step = pl.program_id(0)
    nsteps = pl.num_programs(0)
    ntok = nsteps * _BLK

    z = z_ref[...]
    cb = cb_ref[...]

    z2 = jnp.sum(z * z, axis=1, keepdims=True)
    c2 = jnp.sum(cb * cb, axis=1)
    m = jax.lax.dot_general(z, cb, (((1,), (1,)), ((), ())),
                            preferred_element_type=jnp.float32)
    d = (z2 + c2[None, :]) - 2.0 * m

    dmin = jnp.min(d, axis=1, keepdims=True)
    iota = jax.lax.broadcasted_iota(jnp.int32, (_BLK, _NCODES), 1)
    idx = jnp.min(jnp.where(d == dmin, iota, _NCODES), axis=1,
                  keepdims=True)
    idx_ref[...] = idx

    onehot = (iota == idx).astype(jnp.float32)
    zq = jax.lax.dot_general(onehot, cb, (((1,), (0,)), ((), ())),
                             preferred_element_type=jnp.float32)
    zq_ref[...] = zq

    diff = zq - z
    part_loss = jnp.sum(diff * diff)
    part_counts = jax.lax.dot_general(
        jnp.ones((1, _BLK), jnp.float32), onehot, (((1,), (0,)), ((), ())),
        preferred_element_type=jnp.float32)

    @pl.when(step == 0)
    def _init():
        counts_ref[...] = jnp.zeros_like(counts_ref)
        losssum_ref[...] = jnp.zeros_like(losssum_ref)

    counts_ref[...] += part_counts
    losssum_ref[...] = losssum_ref[...] + part_loss

    @pl.when(step == nsteps - 1)
    def _fin():
        loss_ref[...] = losssum_ref[...] / (ntok * _D)
        p = counts_ref[...] / ntok
        s = jnp.sum(p * jnp.log(p + 1e-10), axis=1, keepdims=True)
        perp_ref[...] = jnp.exp(-s)


def kernel(z_e, codebook):
    shape = z_e.shape
    flat = z_e.reshape(-1, _D)
    ntok = flat.shape[0]
    grid = ntok // _BLK

    zq, idx, loss, perp = pl.pallas_call(
        _vq_body,
        grid=(grid,),
        in_specs=[
            pl.BlockSpec((_BLK, _D), lambda i: (i, 0)),
            pl.BlockSpec((_NCODES, _D), lambda i: (0, 0)),
        ],
        out_specs=[
            pl.BlockSpec((_BLK, _D), lambda i: (i, 0)),
            pl.BlockSpec((_BLK, 1), lambda i: (i, 0)),
            pl.BlockSpec((1, 1), lambda i: (0, 0)),
            pl.BlockSpec((1, 1), lambda i: (0, 0)),
        ],
        out_shape=[
            jax.ShapeDtypeStruct((ntok, _D), jnp.float32),
            jax.ShapeDtypeStruct((ntok, 1), jnp.int32),
            jax.ShapeDtypeStruct((1, 1), jnp.float32),
            jax.ShapeDtypeStruct((1, 1), jnp.float32),
        ],
        scratch_shapes=[
            pltpu.VMEM((1, _NCODES), jnp.float32),
            pltpu.VMEM((1, 1), jnp.float32),
        ],
    )(flat, codebook)

    z_q_st = zq.reshape(shape)
    indices_r = idx[:, 0].reshape(shape[:-1])
    loss_s = loss[0, 0]
    return (z_q_st, indices_r, loss_s, loss_s, perp[0, 0])
